# baseline (speedup 1.0000x reference)
import functools

import jax
import jax.numpy as jnp
from jax import lax
from jax.experimental import pallas as pl
from jax.experimental.pallas import tpu as pltpu
from jax.experimental.pallas import tpu_sc as plsc

D_MODEL = 64
SCALE = float(D_MODEL) ** 0.5
LANES = 16

NC = 2
NS = 16
NW = NC * NS

ROWS = 4096 * 200
RPW = ROWS // NW
G = 128
NG = RPW // G

_mesh = plsc.VectorSubcoreMesh(core_axis_name="c", subcore_axis_name="s")


@functools.partial(
    pl.kernel,
    mesh=_mesh,
    out_type=jax.ShapeDtypeStruct((ROWS, D_MODEL), jnp.float32),
    scratch_types=[
        pltpu.VMEM((NG, G), jnp.int32),
        pltpu.VMEM((G, D_MODEL), jnp.float32),
        pltpu.SemaphoreType.DMA,
    ],
    compiler_params=pltpu.CompilerParams(use_tc_tiling_on_sc=False),
)
def _embed(idx_hbm, table_hbm, out_hbm, idx_v, rows_v, sem):
    wid = lax.axis_index("s") * NC + lax.axis_index("c")
    pltpu.sync_copy(idx_hbm.at[wid], idx_v)
    out_base = wid * RPW

    def step(g, carry):
        pltpu.async_copy(table_hbm.at[idx_v.at[g]], rows_v, sem).wait()

        def scale_row(r, c2):
            for c in range(D_MODEL // LANES):
                sl = pl.ds(c * LANES, LANES)
                rows_v[r, sl] = rows_v[r, sl] * SCALE
            return c2

        lax.fori_loop(0, G, scale_row, 0, unroll=2)
        pltpu.sync_copy(rows_v, out_hbm.at[pl.ds(out_base + g * G, G)])
        return carry

    lax.fori_loop(0, NG, step, 0)


def kernel(x, table):
    idx = x.reshape(NW, NG, G)
    out = _embed(idx, table)
    return out.reshape(x.shape[0], x.shape[1], D_MODEL)


# speedup vs baseline: 1.1624x; 1.1624x over previous
import functools

import jax
import jax.numpy as jnp
from jax import lax
from jax.experimental import pallas as pl
from jax.experimental.pallas import tpu as pltpu
from jax.experimental.pallas import tpu_sc as plsc

D_MODEL = 64
SCALE = float(D_MODEL) ** 0.5
LANES = 16

NC = 2
NS = 16
NW = NC * NS

ROWS = 4096 * 200
RPW = ROWS // NW
G = 128
NG = RPW // G
NB = 8
LEAD = NB // 2
NH = NG // NB

_mesh = plsc.VectorSubcoreMesh(core_axis_name="c", subcore_axis_name="s")


@functools.partial(
    pl.kernel,
    mesh=_mesh,
    out_type=jax.ShapeDtypeStruct((ROWS, D_MODEL), jnp.float32),
    scratch_types=(
        [pltpu.VMEM((NG, G), jnp.int32)]
        + [pltpu.VMEM((G, D_MODEL), jnp.float32) for _ in range(NB)]
        + [pltpu.SemaphoreType.DMA for _ in range(2 * NB)]
    ),
    compiler_params=pltpu.CompilerParams(use_tc_tiling_on_sc=False),
)
def _embed(idx_hbm, table_hbm, out_hbm, idx_v, *bufs):
    rows = bufs[0:NB]
    gsem = bufs[NB:2 * NB]
    osem = bufs[2 * NB:3 * NB]
    wid = lax.axis_index("s") * NC + lax.axis_index("c")
    pltpu.sync_copy(idx_hbm.at[wid], idx_v)
    out_base = wid * RPW

    def gather_start(g, b):
        pltpu.async_copy(table_hbm.at[idx_v.at[g]], rows[b], gsem[b])

    def gather_wait(g, b):
        pltpu.make_async_copy(table_hbm.at[idx_v.at[g]], rows[b], gsem[b]).wait()

    def scatter_start(g, b):
        pltpu.async_copy(rows[b], out_hbm.at[pl.ds(out_base + g * G, G)], osem[b])

    def scatter_wait(b):
        pltpu.make_async_copy(
            rows[b], out_hbm.at[pl.ds(out_base, G)], osem[b]).wait()

    def visit(g, b, pre_fetch, pre_wait):
        b_pre = (b + LEAD) % NB
        gather_wait(g, b)

        def scale_row(r, c2):
            for c in range(D_MODEL // LANES):
                sl = pl.ds(c * LANES, LANES)
                rows[b][r, sl] = rows[b][r, sl] * SCALE
            return c2

        lax.fori_loop(0, G, scale_row, 0, unroll=4)
        scatter_start(g, b)
        if pre_fetch:
            if pre_wait:
                scatter_wait(b_pre)
            gather_start(g + LEAD, b_pre)

    for b in range(LEAD):
        gather_start(b, b)

    for b in range(NB):
        visit(b, b, pre_fetch=True, pre_wait=(b >= LEAD))

    def outer(h, carry):
        for b in range(NB):
            visit(h * NB + b, b, pre_fetch=True, pre_wait=True)
        return carry

    lax.fori_loop(1, NH - 1, outer, 0)

    for b in range(NB):
        visit((NH - 1) * NB + b, b, pre_fetch=(b < LEAD), pre_wait=True)

    for b in range(LEAD, NB):
        scatter_wait(b)


def kernel(x, table):
    idx = x.reshape(NW, NG, G)
    out = _embed(idx, table)
    return out.reshape(x.shape[0], x.shape[1], D_MODEL)
